# baseline (speedup 1.0000x reference)
import functools

import jax
import jax.numpy as jnp
from jax import lax
from jax.experimental import pallas as pl
from jax.experimental.pallas import tpu as pltpu
from jax.experimental.pallas import tpu_sc as plsc

_NBUF = 4


@functools.lru_cache(maxsize=None)
def _make_sc_gather(n_tokens: int, embed: int, chunk: int):
    info = plsc.get_sparse_core_info()
    nw = info.num_cores * info.num_subcores
    assert n_tokens % nw == 0
    per_w = n_tokens // nw
    assert per_w % (_NBUF * chunk) == 0 and chunk <= 128 and chunk % 8 == 0
    n_rounds = per_w // (_NBUF * chunk)
    mesh = plsc.VectorSubcoreMesh(core_axis_name="c", subcore_axis_name="s")

    @functools.partial(
        pl.kernel,
        mesh=mesh,
        out_type=jax.ShapeDtypeStruct((n_tokens, embed), jnp.float32),
        scratch_types=(
            [pltpu.VMEM((per_w,), jnp.int32)]
            + [pltpu.VMEM((chunk, embed), jnp.float32)] * _NBUF
            + [pltpu.SemaphoreType.DMA] * (2 * _NBUF)
        ),
    )
    def gather_k(tab_hbm, idx_hbm, out_hbm, idx_v, *bufs_sems):
        bufs = bufs_sems[:_NBUF]
        gsems = bufs_sems[_NBUF:2 * _NBUF]
        wsems = bufs_sems[2 * _NBUF:]
        wid = lax.axis_index("s") * info.num_cores + lax.axis_index("c")
        base = wid * per_w
        pltpu.sync_copy(idx_hbm.at[pl.ds(base, per_w)], idx_v)

        def start_gather(g, p):
            pltpu.make_async_copy(
                tab_hbm.at[idx_v.at[pl.ds(g * chunk, chunk)]],
                bufs[p], gsems[p]).start()

        def wait_gather(p):
            pltpu.make_async_copy(
                tab_hbm.at[pl.ds(0, chunk)], bufs[p], gsems[p]).wait()

        def start_wb(g, p):
            pltpu.make_async_copy(
                bufs[p], out_hbm.at[pl.ds(base + g * chunk, chunk)],
                wsems[p]).start()

        def wait_wb(p):
            pltpu.make_async_copy(
                bufs[p], out_hbm.at[pl.ds(base, chunk)], wsems[p]).wait()

        for p in range(_NBUF):
            start_gather(p, p)

        def body(t, carry):
            g0 = _NBUF * t
            for p in range(_NBUF):
                wait_gather(p)
                start_wb(g0 + p, p)
            for p in range(_NBUF):
                @pl.when(t < n_rounds - 1)
                def _():
                    wait_wb(p)
                    start_gather(g0 + _NBUF + p, p)
            return carry

        lax.fori_loop(0, n_rounds, body, 0)
        for p in range(_NBUF):
            wait_wb(p)

    return gather_k


def _make_decoder(batch: int, seq: int, embed: int, vocab: int, lb: int,
                  half_rows: int, l_off: int, first: bool):
    assert half_rows % lb == 0
    grid = (half_rows // lb,)
    rows = lb * batch
    blk = l_off // lb

    def body(*refs):
        if first:
            emb_ref, w_ref, b_ref, out_ref = refs
        else:
            _, emb_ref, w_ref, b_ref, out_ref = refs
        y = lax.dot_general(
            w_ref[...], emb_ref[...],
            dimension_numbers=(((1,), (1,)), ((), ())),
            preferred_element_type=jnp.float32,
        ) + b_ref[...]
        for j in range(lb):
            out_ref[j] = y[:, j * batch:(j + 1) * batch]

    in_specs = [
        pl.BlockSpec((rows, embed), lambda i: (i, 0)),
        pl.BlockSpec((vocab, embed), lambda i: (0, 0)),
        pl.BlockSpec((vocab, 1), lambda i: (0, 0)),
    ]
    kwargs = {}
    if not first:
        in_specs = [pl.BlockSpec(memory_space=pl.ANY)] + in_specs
        kwargs["input_output_aliases"] = {0: 0}

    return pl.pallas_call(
        body,
        grid=grid,
        in_specs=in_specs,
        out_specs=pl.BlockSpec((lb, vocab, batch), lambda i: (i + blk, 0, 0)),
        out_shape=jax.ShapeDtypeStruct((seq, vocab, batch), jnp.float32),
        **kwargs,
    )


def kernel(_input, enc_table, dec_w, dec_b):
    b, l = _input.shape
    vocab, embed = dec_w.shape
    idx_lm = _input.T.reshape(-1)
    lh = l // 2
    n_half = b * lh
    bias = dec_b.reshape(-1, 1)
    gather = _make_sc_gather(n_half, embed, 40)
    emb1 = gather(enc_table, idx_lm[:n_half])
    emb2 = gather(enc_table, idx_lm[n_half:])
    out1 = _make_decoder(b, l, embed, vocab, 1, lh, 0, True)(
        emb1, dec_w, bias)
    out2 = _make_decoder(b, l, embed, vocab, 1, lh, lh, False)(
        out1, emb2, dec_w, bias)
    return jnp.transpose(out2, (2, 0, 1))


# speedup vs baseline: 1.0864x; 1.0864x over previous
import functools

import jax
import jax.numpy as jnp
from jax import lax
from jax.experimental import pallas as pl
from jax.experimental.pallas import tpu as pltpu
from jax.experimental.pallas import tpu_sc as plsc

_NBUF = 4


@functools.lru_cache(maxsize=None)
def _make_sc_gather(n_tokens: int, embed: int, chunk: int):
    info = plsc.get_sparse_core_info()
    nw = info.num_cores * info.num_subcores
    assert n_tokens % nw == 0
    per_w = n_tokens // nw
    assert per_w % (_NBUF * chunk) == 0 and chunk <= 128 and chunk % 8 == 0
    n_rounds = per_w // (_NBUF * chunk)
    mesh = plsc.VectorSubcoreMesh(core_axis_name="c", subcore_axis_name="s")

    @functools.partial(
        pl.kernel,
        mesh=mesh,
        out_type=jax.ShapeDtypeStruct((n_tokens, embed), jnp.float32),
        scratch_types=(
            [pltpu.VMEM((per_w,), jnp.int32)]
            + [pltpu.VMEM((chunk, embed), jnp.float32)] * _NBUF
            + [pltpu.SemaphoreType.DMA] * (2 * _NBUF)
        ),
    )
    def gather_k(tab_hbm, idx_hbm, out_hbm, idx_v, *bufs_sems):
        bufs = bufs_sems[:_NBUF]
        gsems = bufs_sems[_NBUF:2 * _NBUF]
        wsems = bufs_sems[2 * _NBUF:]
        wid = lax.axis_index("s") * info.num_cores + lax.axis_index("c")
        base = wid * per_w
        pltpu.sync_copy(idx_hbm.at[pl.ds(base, per_w)], idx_v)

        def start_gather(g, p):
            pltpu.make_async_copy(
                tab_hbm.at[idx_v.at[pl.ds(g * chunk, chunk)]],
                bufs[p], gsems[p]).start()

        def wait_gather(p):
            pltpu.make_async_copy(
                tab_hbm.at[pl.ds(0, chunk)], bufs[p], gsems[p]).wait()

        def start_wb(g, p):
            pltpu.make_async_copy(
                bufs[p], out_hbm.at[pl.ds(base + g * chunk, chunk)],
                wsems[p]).start()

        def wait_wb(p):
            pltpu.make_async_copy(
                bufs[p], out_hbm.at[pl.ds(base, chunk)], wsems[p]).wait()

        for p in range(_NBUF):
            start_gather(p, p)

        def body(t, carry):
            g0 = _NBUF * t
            for p in range(_NBUF):
                wait_gather(p)
                start_wb(g0 + p, p)
            for p in range(_NBUF):
                @pl.when(t < n_rounds - 1)
                def _():
                    wait_wb(p)
                    start_gather(g0 + _NBUF + p, p)
            return carry

        lax.fori_loop(0, n_rounds, body, 0)
        for p in range(_NBUF):
            wait_wb(p)

    return gather_k


def _make_decoder(batch: int, seq: int, embed: int, vocab: int, lb: int,
                  half_rows: int, l_off: int, first: bool):
    assert half_rows % lb == 0
    grid = (half_rows // lb,)
    rows = lb * batch
    blk = l_off // lb

    def body(*refs):
        if first:
            emb_ref, w_ref, b_ref, out_ref = refs
        else:
            _, emb_ref, w_ref, b_ref, out_ref = refs
        y = lax.dot_general(
            w_ref[...], emb_ref[...],
            dimension_numbers=(((1,), (1,)), ((), ())),
            preferred_element_type=jnp.float32,
        ) + b_ref[...]
        for j in range(lb):
            out_ref[j] = y[:, j * batch:(j + 1) * batch]

    in_specs = [
        pl.BlockSpec((rows, embed), lambda i: (i, 0)),
        pl.BlockSpec((vocab, embed), lambda i: (0, 0)),
        pl.BlockSpec((vocab, 1), lambda i: (0, 0)),
    ]
    kwargs = {}
    if not first:
        in_specs = [pl.BlockSpec(memory_space=pl.ANY)] + in_specs
        kwargs["input_output_aliases"] = {0: 0}

    return pl.pallas_call(
        body,
        grid=grid,
        in_specs=in_specs,
        out_specs=pl.BlockSpec((lb, vocab, batch), lambda i: (i + blk, 0, 0)),
        out_shape=jax.ShapeDtypeStruct((seq, vocab, batch), jnp.float32),
        **kwargs,
    )


def kernel(_input, enc_table, dec_w, dec_b):
    b, l = _input.shape
    vocab, embed = dec_w.shape
    idx_lm = _input.T.reshape(-1)
    lh = l // 2
    n_half = b * lh
    bias = dec_b.reshape(-1, 1)
    gather = _make_sc_gather(n_half, embed, 40)
    emb1 = gather(enc_table, idx_lm[:n_half])
    emb2 = gather(enc_table, idx_lm[n_half:])
    out1 = _make_decoder(b, l, embed, vocab, 5, lh, 0, True)(
        emb1, dec_w, bias)
    out2 = _make_decoder(b, l, embed, vocab, 5, lh, lh, False)(
        out1, emb2, dec_w, bias)
    return jnp.transpose(out2, (2, 0, 1))
